# baseline (speedup 1.0000x reference)
import jax
import jax.numpy as jnp
from jax import lax
from jax.experimental import pallas as pl
from jax.experimental.pallas import tpu as pltpu

HEADS = 16
BUCKETS = 64
SEQ = 4096
DIM = 128
TOK = SEQ // BUCKETS
NSPLIT = 4
ROWS = SEQ // NSPLIT
BPC = BUCKETS // NSPLIT


def _body(q_ref, k_ref, qpos_ref, kpos_ref, wq_ref, wk_ref, out_ref,
          mq_s, mk_s):
    j = pl.program_id(1)
    mq_s[pl.ds(j * BPC, BPC)] = jnp.sum(
        q_ref[0].reshape(BPC, TOK, DIM), axis=1) * jnp.float32(1.0 / TOK)
    mk_s[pl.ds(j * BPC, BPC)] = jnp.sum(
        k_ref[0].reshape(BPC, TOK, DIM), axis=1) * jnp.float32(1.0 / TOK)

    @pl.when(j == NSPLIT - 1)
    def _finish():
        wq = wq_ref[0, 0]
        wk = wk_ref[0, 0]
        sq = (jnp.dot(mq_s[...], wq[:DIM], preferred_element_type=jnp.float32)
              + jnp.dot(qpos_ref[0, 0], wq[DIM:],
                        preferred_element_type=jnp.float32))
        sk = (jnp.dot(mk_s[...], wk[:DIM], preferred_element_type=jnp.float32)
              + jnp.dot(kpos_ref[0, 0], wk[DIM:],
                        preferred_element_type=jnp.float32))
        r = lax.dot_general(sq, sk, (((1,), (1,)), ((), ())),
                            preferred_element_type=jnp.float32)
        r = r - jnp.max(r, axis=-1, keepdims=True)
        e = jnp.exp(r)
        out_ref[0] = e / jnp.sum(e, axis=-1, keepdims=True)


def kernel(q, k, q_pos_emb, k_pos_emb, linear_sort_q, linear_sort_k):
    bh = q.shape[0]
    return pl.pallas_call(
        _body,
        grid=(bh, NSPLIT),
        in_specs=[
            pl.BlockSpec((1, ROWS, DIM), lambda i, j: (i, j, 0)),
            pl.BlockSpec((1, ROWS, DIM), lambda i, j: (i, j, 0)),
            pl.BlockSpec((1, 1, BUCKETS, DIM), lambda i, j: (0, i % HEADS, 0, 0)),
            pl.BlockSpec((1, 1, BUCKETS, DIM), lambda i, j: (0, i % HEADS, 0, 0)),
            pl.BlockSpec((1, 1, 2 * DIM, DIM), lambda i, j: (0, i % HEADS, 0, 0)),
            pl.BlockSpec((1, 1, 2 * DIM, DIM), lambda i, j: (0, i % HEADS, 0, 0)),
        ],
        out_specs=pl.BlockSpec((1, BUCKETS, BUCKETS), lambda i, j: (i, 0, 0)),
        out_shape=jax.ShapeDtypeStruct((bh, BUCKETS, BUCKETS), jnp.float32),
        scratch_shapes=[
            pltpu.VMEM((BUCKETS, DIM), jnp.float32),
            pltpu.VMEM((BUCKETS, DIM), jnp.float32),
        ],
        compiler_params=pltpu.CompilerParams(
            dimension_semantics=("parallel", "arbitrary")),
    )(q, k, q_pos_emb, k_pos_emb, linear_sort_q, linear_sort_k)


# speedup vs baseline: 2.3630x; 2.3630x over previous
import jax
import jax.numpy as jnp
from jax import lax
from jax.experimental import pallas as pl
from jax.experimental.pallas import tpu as pltpu

HEADS = 16
BUCKETS = 64
SEQ = 4096
DIM = 128
TOK = SEQ // BUCKETS
SL = 2


def _sortnet(mq, mk, qpos, kpos, wq, wk):
    sq = (jnp.dot(mq, wq[:DIM], preferred_element_type=jnp.float32)
          + jnp.dot(qpos, wq[DIM:], preferred_element_type=jnp.float32))
    sk = (jnp.dot(mk, wk[:DIM], preferred_element_type=jnp.float32)
          + jnp.dot(kpos, wk[DIM:], preferred_element_type=jnp.float32))
    r = lax.dot_general(sq, sk, (((1,), (1,)), ((), ())),
                        preferred_element_type=jnp.float32)
    r = r - jnp.max(r, axis=-1, keepdims=True)
    e = jnp.exp(r)
    return e / jnp.sum(e, axis=-1, keepdims=True)


def _body(q_ref, k_ref, qpos_ref, kpos_ref, wq_ref, wk_ref, out_ref):
    for s in range(SL):
        mq = jnp.sum(q_ref[s].reshape(BUCKETS, TOK, DIM), axis=1) * (
            jnp.float32(1.0 / TOK))
        mk = jnp.sum(k_ref[s].reshape(BUCKETS, TOK, DIM), axis=1) * (
            jnp.float32(1.0 / TOK))
        out_ref[s] = _sortnet(mq, mk, qpos_ref[0, s], kpos_ref[0, s],
                              wq_ref[0, s], wk_ref[0, s])


def kernel(q, k, q_pos_emb, k_pos_emb, linear_sort_q, linear_sort_k):
    bh = q.shape[0]
    n = bh // SL
    return pl.pallas_call(
        _body,
        grid=(n,),
        in_specs=[
            pl.BlockSpec((SL, SEQ, DIM), lambda i: (i, 0, 0)),
            pl.BlockSpec((SL, SEQ, DIM), lambda i: (i, 0, 0)),
            pl.BlockSpec((1, SL, BUCKETS, DIM),
                         lambda i: (0, i % (HEADS // SL), 0, 0)),
            pl.BlockSpec((1, SL, BUCKETS, DIM),
                         lambda i: (0, i % (HEADS // SL), 0, 0)),
            pl.BlockSpec((1, SL, 2 * DIM, DIM),
                         lambda i: (0, i % (HEADS // SL), 0, 0)),
            pl.BlockSpec((1, SL, 2 * DIM, DIM),
                         lambda i: (0, i % (HEADS // SL), 0, 0)),
        ],
        out_specs=pl.BlockSpec((SL, BUCKETS, BUCKETS), lambda i: (i, 0, 0)),
        out_shape=jax.ShapeDtypeStruct((bh, BUCKETS, BUCKETS), jnp.float32),
    )(q, k, q_pos_emb, k_pos_emb, linear_sort_q, linear_sort_k)
